# baseline (speedup 1.0000x reference)
import functools

import jax
import jax.numpy as jnp
from jax import lax
from jax.experimental import pallas as pl
from jax.experimental.pallas import tpu as pltpu
from jax.experimental.pallas import tpu_sc as plsc

_VOCAB = 100000
_DIM = 64
_ROW = 128
_PITCH = 129
_HALF = 50048
_BATCH = 16384
_NC = 2
_NS = 16
_L = 16
_NW = _NC * _NS
_BPW = _BATCH // _NW
_CHUNK = 256
_PHASES = _BPW // _CHUNK
_GROUPS = _CHUNK // _L
_NBLK = _HALF // _ROW
_BLKW = 13


def _transpose_halfblocks(inva, invb, outv, lid):
    def jc_step(jc, carry):
        j0 = jc * _L
        for jj in range(_L):
            j = j0 + jj
            jva = jnp.full((_L,), 0, jnp.int32) + j
            jvb = jva + _DIM
            for c in range(_ROW // _L):
                rowv = lid + c * _L
                va = inva[j, pl.ds(c * _L, _L)]
                vb = invb[j, pl.ds(c * _L, _L)]
                plsc.store_scatter(outv, [rowv, jva], va)
                plsc.store_scatter(outv, [rowv, jvb], vb)
        return carry

    lax.fori_loop(0, _DIM // _L, jc_step, 0)


def _tr_body(hT_hbm, tT_hbm, hscr_hbm, tscr_hbm,
             inva0, invb0, inva1, invb1, outv0, outv1,
             sia0, sib0, sia1, sib1, so0, so1):
    wid = lax.axis_index("s") * _NC + lax.axis_index("c")
    base_blk = wid * _BLKW
    lid = lax.iota(jnp.int32, _L)

    def start_in(bk, inva, invb, sia, sib, table_hbm):
        c0 = bk * _ROW
        pltpu.async_copy(table_hbm.at[:, pl.ds(c0, _ROW)], inva, sia)
        pltpu.async_copy(table_hbm.at[:, pl.ds(c0 + _HALF, _ROW)], invb, sib)

    def wait_in(bk, inva, invb, sia, sib, table_hbm):
        c0 = bk * _ROW
        pltpu.make_async_copy(
            table_hbm.at[:, pl.ds(c0, _ROW)], inva, sia).wait()
        pltpu.make_async_copy(
            table_hbm.at[:, pl.ds(c0 + _HALF, _ROW)], invb, sib).wait()

    def wait_out(outv, so, scr_hbm):
        pltpu.make_async_copy(
            outv.at[:, pl.ds(0, _ROW)],
            scr_hbm.at[pl.ds(0, _ROW), pl.ds(0, _ROW)], so).wait()

    @pl.when(base_blk < _NBLK)
    def _():
        start_in(base_blk, inva0, invb0, sia0, sib0, hT_hbm)

    def step(i2, carry):
        bk = base_blk + i2
        v = bk < _NBLK
        vprev = (i2 >= 1) & (bk - 1 < _NBLK)
        vnext = (i2 + 1 < _BLKW) & (bk + 1 < _NBLK)

        @pl.when(v)
        def _():
            wait_in(bk, inva0, invb0, sia0, sib0, hT_hbm)
            start_in(bk, inva1, invb1, sia1, sib1, tT_hbm)

        @pl.when(vprev)
        def _():
            wait_out(outv0, so0, hscr_hbm)

        @pl.when(v)
        def _():
            _transpose_halfblocks(inva0, invb0, outv0, lid)
            pltpu.async_copy(
                outv0.at[:, pl.ds(0, _ROW)],
                hscr_hbm.at[pl.ds(bk * _ROW, _ROW), pl.ds(0, _ROW)], so0)

        @pl.when(v)
        def _():
            wait_in(bk, inva1, invb1, sia1, sib1, tT_hbm)

        @pl.when(vnext)
        def _():
            start_in(bk + 1, inva0, invb0, sia0, sib0, hT_hbm)

        @pl.when(vprev)
        def _():
            wait_out(outv1, so1, tscr_hbm)

        @pl.when(v)
        def _():
            _transpose_halfblocks(inva1, invb1, outv1, lid)
            pltpu.async_copy(
                outv1.at[:, pl.ds(0, _ROW)],
                tscr_hbm.at[pl.ds(bk * _ROW, _ROW), pl.ds(0, _ROW)], so1)

        return carry

    lax.fori_loop(0, _BLKW, step, 0)

    @pl.when(base_blk + _BLKW - 1 < _NBLK)
    def _():
        wait_out(outv0, so0, hscr_hbm)
        wait_out(outv1, so1, tscr_hbm)


_tr_kernel = functools.partial(
    pl.kernel,
    out_type=(jax.ShapeDtypeStruct((_HALF, _ROW), jnp.float32),
              jax.ShapeDtypeStruct((_HALF, _ROW), jnp.float32)),
    mesh=plsc.VectorSubcoreMesh(core_axis_name="c", subcore_axis_name="s",
                                num_cores=_NC, num_subcores=_NS),
    compiler_params=pltpu.CompilerParams(needs_layout_passes=False),
    scratch_types=[
        pltpu.VMEM((_DIM, _ROW), jnp.float32),
        pltpu.VMEM((_DIM, _ROW), jnp.float32),
        pltpu.VMEM((_DIM, _ROW), jnp.float32),
        pltpu.VMEM((_DIM, _ROW), jnp.float32),
        pltpu.VMEM((_ROW, _PITCH), jnp.float32),
        pltpu.VMEM((_ROW, _PITCH), jnp.float32),
        pltpu.SemaphoreType.DMA,
        pltpu.SemaphoreType.DMA,
        pltpu.SemaphoreType.DMA,
        pltpu.SemaphoreType.DMA,
        pltpu.SemaphoreType.DMA,
        pltpu.SemaphoreType.DMA,
    ],
)(_tr_body)


def _dot_body(srow_hbm, drow_hbm, soff_hbm, doff_hbm, head_hbm, tail_hbm,
              out_hbm, srow_v, drow_v, soff_v, doff_v, hrows_v, trows_v,
              out_v, sem_h, sem_t):
    wid = lax.axis_index("s") * _NC + lax.axis_index("c")
    base = wid * _BPW
    pltpu.sync_copy(srow_hbm.at[pl.ds(base, _BPW)], srow_v)
    pltpu.sync_copy(drow_hbm.at[pl.ds(base, _BPW)], drow_v)
    pltpu.sync_copy(soff_hbm.at[pl.ds(base, _BPW)], soff_v)
    pltpu.sync_copy(doff_hbm.at[pl.ds(base, _BPW)], doff_v)

    lid = lax.iota(jnp.int32, _L)

    def phase_step(p, carry):
        poff = p * _CHUNK
        ch = pltpu.async_copy(head_hbm.at[srow_v.at[pl.ds(poff, _CHUNK)]],
                              hrows_v, sem_h)
        ct = pltpu.async_copy(tail_hbm.at[drow_v.at[pl.ds(poff, _CHUNK)]],
                              trows_v, sem_t)
        ch.wait()
        ct.wait()

        def group_step(g, inner):
            pbase = g * _L
            soffs = soff_v[pl.ds(poff + pbase, _L)]
            doffs = doff_v[pl.ds(poff + pbase, _L)]
            res = jnp.zeros((_L,), jnp.float32)
            for p_ in range(_L):
                row = pbase + p_
                ho = soffs[p_]
                to = doffs[p_]
                prod = (hrows_v[row, pl.ds(ho, _L)]
                        * trows_v[row, pl.ds(to, _L)]
                        + hrows_v[row, pl.ds(ho + _L, _L)]
                        * trows_v[row, pl.ds(to + _L, _L)]
                        + hrows_v[row, pl.ds(ho + 2 * _L, _L)]
                        * trows_v[row, pl.ds(to + 2 * _L, _L)]
                        + hrows_v[row, pl.ds(ho + 3 * _L, _L)]
                        * trows_v[row, pl.ds(to + 3 * _L, _L)])
                res = jnp.where(lid == p_, jnp.sum(prod), res)
            out_v[pl.ds(poff + pbase, _L)] = 1.0 / (1.0 + jnp.exp(res))
            return inner

        lax.fori_loop(0, _GROUPS, group_step, 0)
        return carry

    lax.fori_loop(0, _PHASES, phase_step, 0)
    pltpu.sync_copy(out_v, out_hbm.at[pl.ds(base, _BPW)])


_dot_kernel = functools.partial(
    pl.kernel,
    out_type=jax.ShapeDtypeStruct((_BATCH,), jnp.float32),
    mesh=plsc.VectorSubcoreMesh(core_axis_name="c", subcore_axis_name="s",
                                num_cores=_NC, num_subcores=_NS),
    compiler_params=pltpu.CompilerParams(needs_layout_passes=False),
    scratch_types=[
        pltpu.VMEM((_BPW,), jnp.int32),
        pltpu.VMEM((_BPW,), jnp.int32),
        pltpu.VMEM((_BPW,), jnp.int32),
        pltpu.VMEM((_BPW,), jnp.int32),
        pltpu.VMEM((_CHUNK, _ROW), jnp.float32),
        pltpu.VMEM((_CHUNK, _ROW), jnp.float32),
        pltpu.VMEM((_BPW,), jnp.float32),
        pltpu.SemaphoreType.DMA,
        pltpu.SemaphoreType.DMA,
    ],
)(_dot_body)


def kernel(x, emb_head, emb_tail):
    s = x[:, 0]
    d = x[:, 1]
    s_hi = (s >= _HALF).astype(jnp.int32)
    d_hi = (d >= _HALF).astype(jnp.int32)
    s_row = s - s_hi * _HALF
    d_row = d - d_hi * _HALF
    s_off = s_hi * _DIM
    d_off = d_hi * _DIM
    hscr, tscr = _tr_kernel(emb_head.T, emb_tail.T)
    return _dot_kernel(s_row, d_row, s_off, d_off, hscr, tscr)


# speedup vs baseline: 2.1126x; 2.1126x over previous
import functools

import jax
import jax.numpy as jnp
from jax import lax
from jax.experimental import pallas as pl
from jax.experimental.pallas import tpu as pltpu
from jax.experimental.pallas import tpu_sc as plsc

_VOCAB = 100000
_DIM = 64
_BATCH = 16384
_NC = 2
_NS = 16
_L = 16
_NW = _NC * _NS
_BPW = _BATCH // _NW
_GROUPS = _BPW // _L


def _sc_body(s_hbm, d_hbm, head_hbm, tail_hbm, out_hbm,
             s_v, d_v, hrows_v, trows_v, out_v, sem_h, sem_t):
    wid = lax.axis_index("s") * _NC + lax.axis_index("c")
    base = wid * _BPW
    pltpu.sync_copy(s_hbm.at[pl.ds(base, _BPW)], s_v)
    pltpu.sync_copy(d_hbm.at[pl.ds(base, _BPW)], d_v)
    ch = pltpu.async_copy(head_hbm.at[s_v], hrows_v, sem_h)
    ct = pltpu.async_copy(tail_hbm.at[d_v], trows_v, sem_t)
    ch.wait()
    ct.wait()

    lid = lax.iota(jnp.int32, _L)

    def group_step(g, carry):
        pbase = g * _L
        res = jnp.zeros((_L,), jnp.float32)
        for p in range(_L):
            row = pbase + p
            prod = (hrows_v[row, pl.ds(0, _L)] * trows_v[row, pl.ds(0, _L)]
                    + hrows_v[row, pl.ds(_L, _L)] * trows_v[row, pl.ds(_L, _L)]
                    + hrows_v[row, pl.ds(2 * _L, _L)] * trows_v[row, pl.ds(2 * _L, _L)]
                    + hrows_v[row, pl.ds(3 * _L, _L)] * trows_v[row, pl.ds(3 * _L, _L)])
            res = jnp.where(lid == p, jnp.sum(prod), res)
        out_v[pl.ds(pbase, _L)] = 1.0 / (1.0 + jnp.exp(res))
        return carry

    lax.fori_loop(0, _GROUPS, group_step, 0)
    pltpu.sync_copy(out_v, out_hbm.at[pl.ds(base, _BPW)])


_sc_kernel = functools.partial(
    pl.kernel,
    out_type=jax.ShapeDtypeStruct((_BATCH,), jnp.float32),
    mesh=plsc.VectorSubcoreMesh(core_axis_name="c", subcore_axis_name="s",
                                num_cores=_NC, num_subcores=_NS),
    compiler_params=pltpu.CompilerParams(needs_layout_passes=False,
                                         use_tc_tiling_on_sc=False),
    scratch_types=[
        pltpu.VMEM((_BPW,), jnp.int32),
        pltpu.VMEM((_BPW,), jnp.int32),
        pltpu.VMEM((_BPW, _DIM), jnp.float32),
        pltpu.VMEM((_BPW, _DIM), jnp.float32),
        pltpu.VMEM((_BPW,), jnp.float32),
        pltpu.SemaphoreType.DMA,
        pltpu.SemaphoreType.DMA,
    ],
)(_sc_body)


def kernel(x, emb_head, emb_tail):
    s = x[:, 0]
    d = x[:, 1]
    return _sc_kernel(s, d, emb_head, emb_tail)
